# baseline (speedup 1.0000x reference)
import functools

import jax
import jax.numpy as jnp
from jax import lax
from jax.experimental import pallas as pl
from jax.experimental.pallas import tpu as pltpu
from jax.experimental.pallas import tpu_sc as plsc

EMBEDDING_DIM = 32
BATCH = 16384
NROWS = 1000000
LANES = 128
TILE_COLS = 7813
FULL_T = 7812
TAIL_I0 = FULL_T * LANES
TAIL_N = NROWS - TAIL_I0
TILE_WORDS = EMBEDDING_DIM * LANES
APP0 = FULL_T * TILE_WORDS
DEPTH = 16


def _memcpy_kernel(info, mesh):
    @functools.partial(
        pl.kernel,
        mesh=mesh,
        out_type=jax.ShapeDtypeStruct(
            (TILE_COLS, EMBEDDING_DIM, LANES), jnp.float32
        ),
        scratch_types=[pltpu.SemaphoreType.DMA],
    )
    def k1(table_t_hbm, tail_app_hbm, raw_hbm, sem):
        wid = lax.axis_index("s") * info.num_cores + lax.axis_index("c")
        nw = info.num_cores * info.num_subcores

        def body(j, _):
            t = wid + j * nw
            @pl.when(j >= DEPTH)
            def _():
                pltpu.make_async_copy(
                    table_t_hbm.at[:, pl.ds(0, LANES)],
                    raw_hbm.at[0],
                    sem,
                ).wait()
            pltpu.async_copy(
                table_t_hbm.at[:, pl.ds(t * LANES, LANES)],
                raw_hbm.at[t],
                sem,
            )
            return ()

        n_j = (FULL_T - wid + nw - 1) // nw
        lax.fori_loop(0, n_j, body, ())

        def drain(j, _):
            pltpu.make_async_copy(
                table_t_hbm.at[:, pl.ds(0, LANES)],
                raw_hbm.at[0],
                sem,
            ).wait()
            return ()

        lax.fori_loop(0, jnp.minimum(n_j, DEPTH), drain, ())

        @pl.when(wid == 0)
        def _():
            pltpu.sync_copy(
                tail_app_hbm, raw_hbm.at[FULL_T].at[pl.ds(0, 16), :]
            )

    return k1


def _gather_kernel(info, mesh, b_per_w):
    n_vregs = b_per_w // 16

    @functools.partial(
        pl.kernel,
        mesh=mesh,
        out_type=jax.ShapeDtypeStruct((EMBEDDING_DIM, BATCH), jnp.float32),
        scratch_types=[
            pltpu.VMEM((b_per_w,), jnp.int32),
            pltpu.VMEM((EMBEDDING_DIM * b_per_w,), jnp.int32),
            pltpu.VMEM((EMBEDDING_DIM * b_per_w,), jnp.float32),
            pltpu.SemaphoreType.DMA,
        ],
        compiler_params=pltpu.CompilerParams(use_tc_tiling_on_sc=False),
    )
    def k2(x_hbm, flat_hbm, out_t_hbm, xv, offs_v, rows_v, sem):
        wid = lax.axis_index("s") * info.num_cores + lax.axis_index("c")
        base = wid * b_per_w
        pltpu.sync_copy(x_hbm.at[pl.ds(base, b_per_w)], xv)

        def offs_body(jv, _):
            xq = xv[pl.ds(jv * 16, 16)]
            bad = xq >= TAIL_I0
            q = (xq >> 7) * TILE_WORDS + (xq & 127)
            bbq = (xq - TAIL_I0) * EMBEDDING_DIM + APP0
            for d in range(EMBEDDING_DIM):
                woff = jnp.where(bad, bbq + d, q + d * LANES)
                offs_v[pl.ds(d * b_per_w + jv * 16, 16)] = woff
            return ()

        lax.fori_loop(0, n_vregs, offs_body, ())

        pltpu.async_copy(flat_hbm.at[offs_v], rows_v, sem).wait()

        for d in range(EMBEDDING_DIM):
            pltpu.sync_copy(
                rows_v.at[pl.ds(d * b_per_w, b_per_w)],
                out_t_hbm.at[d].at[pl.ds(base, b_per_w)],
            )

    return k2


def kernel(x, table):
    info = plsc.get_sparse_core_info()
    nw = info.num_cores * info.num_subcores
    b_per_w = BATCH // nw

    mesh = plsc.VectorSubcoreMesh(core_axis_name="c", subcore_axis_name="s")

    tail_app = jnp.reshape(
        lax.slice(table, (TAIL_I0, 0), (NROWS, EMBEDDING_DIM)),
        (TAIL_N * EMBEDDING_DIM // LANES, LANES),
    )
    raw = _memcpy_kernel(info, mesh)(table.T, tail_app)
    flat = jnp.reshape(raw, (-1,))
    out_t = _gather_kernel(info, mesh, b_per_w)(x, flat)
    return out_t.T


# speedup vs baseline: 28.7151x; 28.7151x over previous
import functools

import jax
import jax.numpy as jnp
from jax import lax
from jax.experimental import pallas as pl
from jax.experimental.pallas import tpu as pltpu
from jax.experimental.pallas import tpu_sc as plsc

EMBEDDING_DIM = 32
BATCH = 16384
NROWS = 1000000
LANES = 128
TILE_COLS = 7813
FULL_T = 7812
TAIL_I0 = FULL_T * LANES
TAIL_N = NROWS - TAIL_I0
TILE_WORDS = EMBEDDING_DIM * LANES
APP0 = FULL_T * TILE_WORDS
DEPTH = 16


def _memcpy_kernel(info, mesh):
    SLAB = 4
    N_SLABS = 244 // SLAB

    @functools.partial(
        pl.kernel,
        mesh=mesh,
        out_type=jax.ShapeDtypeStruct(
            (TILE_COLS, EMBEDDING_DIM, LANES), jnp.float32
        ),
        scratch_types=[
            pltpu.VMEM((2, EMBEDDING_DIM, SLAB * LANES), jnp.float32),
            pltpu.SemaphoreType.DMA,
            pltpu.SemaphoreType.DMA,
        ],
    )
    def k1(table_t_hbm, tail_app_hbm, raw_hbm, buf, sem_r, sem_w):
        wid = lax.axis_index("s") * info.num_cores + lax.axis_index("c")
        t0 = wid * 244 + jnp.minimum(wid, 4)

        def read_slab(s, p):
            pltpu.async_copy(
                table_t_hbm.at[:, pl.ds((t0 + s * SLAB) * LANES, SLAB * LANES)],
                buf.at[p],
                sem_r,
            )

        read_slab(0, 0)

        def body(s, _):
            p = s & 1
            @pl.when(s >= 1)
            def _():
                for _i in range(SLAB):
                    pltpu.make_async_copy(
                        buf.at[0].at[:, pl.ds(0, LANES)],
                        raw_hbm.at[0],
                        sem_w,
                    ).wait()
            @pl.when(s + 1 < N_SLABS)
            def _():
                read_slab(s + 1, 1 - p)
            pltpu.make_async_copy(
                table_t_hbm.at[:, pl.ds(0, SLAB * LANES)],
                buf.at[0],
                sem_r,
            ).wait()
            for i in range(SLAB):
                pltpu.async_copy(
                    buf.at[p].at[:, pl.ds(i * LANES, LANES)],
                    raw_hbm.at[t0 + s * SLAB + i],
                    sem_w,
                )
            return ()

        lax.fori_loop(0, N_SLABS, body, ())
        for _i in range(SLAB):
            pltpu.make_async_copy(
                buf.at[0].at[:, pl.ds(0, LANES)],
                raw_hbm.at[0],
                sem_w,
            ).wait()

        @pl.when(wid < 4)
        def _():
            t_x = t0 + 244
            pltpu.sync_copy(
                table_t_hbm.at[:, pl.ds(t_x * LANES, LANES)],
                buf.at[0].at[:, pl.ds(0, LANES)],
            )
            pltpu.sync_copy(buf.at[0].at[:, pl.ds(0, LANES)], raw_hbm.at[t_x])

        @pl.when(wid == 0)
        def _():
            pltpu.sync_copy(
                tail_app_hbm, raw_hbm.at[FULL_T].at[pl.ds(0, 16), :]
            )

    return k1


def _gather_kernel(info, mesh, b_per_w):
    n_vregs = b_per_w // 16

    @functools.partial(
        pl.kernel,
        mesh=mesh,
        out_type=jax.ShapeDtypeStruct((EMBEDDING_DIM, BATCH), jnp.float32),
        scratch_types=[
            pltpu.VMEM((b_per_w,), jnp.int32),
            pltpu.VMEM((EMBEDDING_DIM * b_per_w,), jnp.int32),
            pltpu.VMEM((EMBEDDING_DIM * b_per_w,), jnp.float32),
            pltpu.SemaphoreType.DMA,
        ],
        compiler_params=pltpu.CompilerParams(use_tc_tiling_on_sc=False),
    )
    def k2(x_hbm, flat_hbm, out_t_hbm, xv, offs_v, rows_v, sem):
        wid = lax.axis_index("s") * info.num_cores + lax.axis_index("c")
        base = wid * b_per_w
        pltpu.sync_copy(x_hbm.at[pl.ds(base, b_per_w)], xv)

        def offs_body(jv, _):
            xq = xv[pl.ds(jv * 16, 16)]
            bad = xq >= TAIL_I0
            q = (xq >> 7) * TILE_WORDS + (xq & 127)
            bbq = (xq - TAIL_I0) * EMBEDDING_DIM + APP0
            for d in range(EMBEDDING_DIM):
                woff = jnp.where(bad, bbq + d, q + d * LANES)
                offs_v[pl.ds(d * b_per_w + jv * 16, 16)] = woff
            return ()

        lax.fori_loop(0, n_vregs, offs_body, ())

        pltpu.async_copy(flat_hbm.at[offs_v], rows_v, sem).wait()

        for d in range(EMBEDDING_DIM):
            pltpu.sync_copy(
                rows_v.at[pl.ds(d * b_per_w, b_per_w)],
                out_t_hbm.at[d].at[pl.ds(base, b_per_w)],
            )

    return k2


def kernel(x, table):
    info = plsc.get_sparse_core_info()
    nw = info.num_cores * info.num_subcores
    b_per_w = BATCH // nw

    mesh = plsc.VectorSubcoreMesh(core_axis_name="c", subcore_axis_name="s")

    tail_app = jnp.reshape(
        lax.slice(table, (TAIL_I0, 0), (NROWS, EMBEDDING_DIM)),
        (TAIL_N * EMBEDDING_DIM // LANES, LANES),
    )
    raw = _memcpy_kernel(info, mesh)(table.T, tail_app)
    flat = jnp.reshape(raw, (-1,))
    out_t = _gather_kernel(info, mesh, b_per_w)(x, flat)
    return out_t.T
